# baseline (speedup 1.0000x reference)
import functools

import jax
import jax.numpy as jnp
import numpy as np
from jax import lax
from jax.experimental import pallas as pl
from jax.experimental.pallas import tpu as pltpu
from jax.experimental.pallas import tpu_sc as plsc

LANES = 16


@functools.lru_cache(maxsize=None)
def _make_gather_sum(seq, batch, vocab, dim):
    info = plsc.get_sparse_core_info()
    nc, ns = info.num_cores, info.num_subcores
    nw = nc * ns
    assert batch % nw == 0
    bpw = batch // nw
    words = dim // 2
    ngrp = dim // (2 * LANES)
    assert seq % 2 == 0

    mesh = plsc.VectorSubcoreMesh(core_axis_name="c", subcore_axis_name="s")

    @functools.partial(
        pl.kernel,
        mesh=mesh,
        out_type=jax.ShapeDtypeStruct((batch, dim), jnp.float32),
        scratch_types=[
            pltpu.VMEM((seq, bpw), jnp.int32),
            pltpu.VMEM((2, bpw, words), jnp.int32),
            pltpu.VMEM((bpw, dim), jnp.float32),
            pltpu.SemaphoreType.DMA,
            pltpu.SemaphoreType.DMA,
        ],
        compiler_params=pltpu.CompilerParams(
            needs_layout_passes=False, use_tc_tiling_on_sc=False),
    )
    def gather_sum(idx_hbm, table_hbm, out_hbm, idx_v, rows_v, acc_v,
                   sem0, sem1):
        wid = lax.axis_index("s") * nc + lax.axis_index("c")
        base = wid * bpw

        pltpu.sync_copy(idx_hbm.at[:, pl.ds(base, bpw)], idx_v)

        def zrow(j, carry):
            for v in range(dim // LANES):
                acc_v[j, pl.ds(v * LANES, LANES)] = jnp.zeros(
                    (LANES,), jnp.float32)
            return carry

        lax.fori_loop(0, bpw, zrow, 0, unroll=8)

        sems = (sem0, sem1)

        def issue(s, b):
            pltpu.async_copy(table_hbm.at[idx_v.at[s]], rows_v.at[b],
                             sems[b])

        def wait(b):
            pltpu.make_async_copy(table_hbm.at[pl.ds(0, bpw)],
                                  rows_v.at[b], sems[b]).wait()

        def accum(b):
            def arow(j, carry):
                for v in range(ngrp):
                    x = rows_v[b, j, pl.ds(v * LANES, LANES)]
                    lo, hi = plsc.unpack(
                        plsc.bitcast(x, jnp.bfloat16),
                        format=plsc.PackFormat.INTERLEAVED)
                    plsc.addupdate(
                        acc_v.at[j, pl.ds(2 * v * LANES, LANES)], lo)
                    plsc.addupdate(
                        acc_v.at[j, pl.ds((2 * v + 1) * LANES, LANES)], hi)
                return carry

            lax.fori_loop(0, bpw, arow, 0, unroll=4)

        issue(0, 0)

        def sbody(i, carry):
            s0 = 2 * i
            issue(s0 + 1, 1)
            wait(0)
            accum(0)

            @pl.when(s0 + 2 < seq)
            def _():
                issue(s0 + 2, 0)

            wait(1)
            accum(1)
            return carry

        lax.fori_loop(0, seq // 2, sbody, 0)

        pltpu.sync_copy(acc_v, out_hbm.at[pl.ds(base, bpw)])

    return gather_sum


def _fc_body(x_ref, w_ref, b_ref, o_ref):
    o_ref[...] = lax.dot_general(
        x_ref[...], w_ref[...], (((1,), (1,)), ((), ())),
        preferred_element_type=jnp.float32) + b_ref[...]


@functools.lru_cache(maxsize=None)
def _make_fc(batch, dim, out_dim):
    blk = min(batch, 512)
    return pl.pallas_call(
        _fc_body,
        grid=(batch // blk,),
        in_specs=[
            pl.BlockSpec((blk, dim), lambda i: (i, 0)),
            pl.BlockSpec((out_dim, dim), lambda i: (0, 0)),
            pl.BlockSpec((1, out_dim), lambda i: (0, 0)),
        ],
        out_specs=pl.BlockSpec((blk, out_dim), lambda i: (i, 0)),
        out_shape=jax.ShapeDtypeStruct((batch, out_dim), jnp.float32),
    )


@functools.lru_cache(maxsize=None)
def _interleave_perm(dim):
    perm = np.zeros((dim,), np.int32)
    for g in range(dim // (2 * LANES)):
        for k in range(LANES):
            perm[32 * g + k] = 32 * g + 2 * k
            perm[32 * g + LANES + k] = 32 * g + 2 * k + 1
    return perm


def kernel(text, embedding_table, fc_weight, fc_bias):
    seq, batch = text.shape
    vocab, dim = embedding_table.shape
    out_dim = fc_weight.shape[0]

    idx = text.astype(jnp.int32)
    packed = lax.bitcast_convert_type(
        embedding_table.astype(jnp.bfloat16).reshape(vocab, dim // 2, 2),
        jnp.int32)
    summed = _make_gather_sum(seq, batch, vocab, dim)(idx, packed)
    w_perm = fc_weight[:, _interleave_perm(dim)]
    fc = _make_fc(batch, dim, out_dim)
    return fc(summed, w_perm, fc_bias.reshape(1, out_dim))


# speedup vs baseline: 3.6768x; 3.6768x over previous
import functools

import jax
import jax.numpy as jnp
from jax import lax
from jax.experimental import pallas as pl
from jax.experimental.pallas import tpu as pltpu
from jax.experimental.pallas import tpu_sc as plsc

LANES = 16
NBUF = 4


@functools.lru_cache(maxsize=None)
def _make_gather_sum(seq, batch, vocab, dim):
    info = plsc.get_sparse_core_info()
    nc, ns = info.num_cores, info.num_subcores
    nw = nc * ns
    assert batch % nw == 0
    bpw = batch // nw
    vpr = dim // LANES
    assert seq % NBUF == 0

    mesh = plsc.VectorSubcoreMesh(core_axis_name="c", subcore_axis_name="s")

    @functools.partial(
        pl.kernel,
        mesh=mesh,
        out_type=jax.ShapeDtypeStruct((batch, dim), jnp.float32),
        scratch_types=[
            pltpu.VMEM((seq, bpw), jnp.int32),
            pltpu.VMEM((NBUF, bpw, dim), jnp.float32),
            pltpu.VMEM((bpw, dim), jnp.float32),
        ] + [pltpu.SemaphoreType.DMA] * NBUF,
    )
    def gather_sum(idx_hbm, table_hbm, out_hbm, idx_v, rows_v, acc_v,
                   *sems):
        wid = lax.axis_index("s") * nc + lax.axis_index("c")
        base = wid * bpw

        pltpu.sync_copy(idx_hbm.at[:, pl.ds(base, bpw)], idx_v)

        def zrow(j, carry):
            for v in range(vpr):
                acc_v[j, pl.ds(v * LANES, LANES)] = jnp.zeros(
                    (LANES,), jnp.float32)
            return carry

        lax.fori_loop(0, bpw, zrow, 0, unroll=8)

        def issue(s, b):
            pltpu.async_copy(table_hbm.at[idx_v.at[s]], rows_v.at[b],
                             sems[b])

        def wait(b):
            pltpu.make_async_copy(table_hbm.at[pl.ds(0, bpw)],
                                  rows_v.at[b], sems[b]).wait()

        def accum(b):
            def arow(j, carry):
                for v in range(vpr):
                    sl = pl.ds(v * LANES, LANES)
                    plsc.addupdate(acc_v.at[j, sl], rows_v[b, j, sl])
                return carry

            lax.fori_loop(0, bpw, arow, 0, unroll=4)

        for b in range(NBUF - 1):
            issue(b, b)

        def sbody(i, carry):
            s0 = NBUF * i
            for b in range(NBUF):
                s = s0 + b
                nxt = s + NBUF - 1

                @pl.when(nxt < seq)
                def _():
                    issue(nxt, (b + NBUF - 1) % NBUF)

                wait(b)
                accum(b)
            return carry

        lax.fori_loop(0, seq // NBUF, sbody, 0)

        pltpu.sync_copy(acc_v, out_hbm.at[pl.ds(base, bpw)])

    return gather_sum


def _fc_body(x_ref, w_ref, b_ref, o_ref):
    o_ref[...] = lax.dot_general(
        x_ref[...], w_ref[...], (((1,), (1,)), ((), ())),
        preferred_element_type=jnp.float32) + b_ref[...]


@functools.lru_cache(maxsize=None)
def _make_fc(batch, dim, out_dim):
    blk = min(batch, 512)
    return pl.pallas_call(
        _fc_body,
        grid=(batch // blk,),
        in_specs=[
            pl.BlockSpec((blk, dim), lambda i: (i, 0)),
            pl.BlockSpec((out_dim, dim), lambda i: (0, 0)),
            pl.BlockSpec((1, out_dim), lambda i: (0, 0)),
        ],
        out_specs=pl.BlockSpec((blk, out_dim), lambda i: (i, 0)),
        out_shape=jax.ShapeDtypeStruct((batch, out_dim), jnp.float32),
    )


def kernel(text, embedding_table, fc_weight, fc_bias):
    seq, batch = text.shape
    vocab, dim = embedding_table.shape
    out_dim = fc_weight.shape[0]

    idx = text.astype(jnp.int32)
    summed = _make_gather_sum(seq, batch, vocab, dim)(idx, embedding_table)
    fc = _make_fc(batch, dim, out_dim)
    return fc(summed, fc_weight, fc_bias.reshape(1, out_dim))
